# Initial kernel scaffold; baseline (speedup 1.0000x reference)
import jax
import jax.numpy as jnp
from jax.experimental import pallas as pl


def kernel(ind_2, px, ix):
    raise NotImplementedError("write your pallas kernel here")


# speedup vs baseline: 81.2561x; 81.2561x over previous
import functools

import jax
import jax.numpy as jnp
from jax import lax
from jax.experimental import pallas as pl
from jax.experimental.pallas import tpu as pltpu
from jax.experimental.pallas import tpu_sc as plsc

N_PAIRS = 1_600_000
N_ATOMS = 50_000
ROW = 48

NC = 2
NS = 16
HALF = N_ATOMS // NC
DUMMY = HALF

B = 1024
SUB = 128
NSUB = B // SUB
E_TILE = N_PAIRS // NS
NBLK = -(-E_TILE // B)
ACC_ROWS = 25_216
ZROWS = ACC_ROWS // NS
W = 1568

_mesh = plsc.VectorSubcoreMesh(
    core_axis_name="c", subcore_axis_name="s", num_cores=NC, num_subcores=NS)


@functools.partial(
    pl.kernel,
    out_type=jax.ShapeDtypeStruct((N_ATOMS, ROW), jnp.float32),
    mesh=_mesh,
    scratch_types=[
        pltpu.VMEM((B,), jnp.int32),
        pltpu.VMEM((B, ROW), jnp.float32),
        pltpu.VMEM((NSUB, SUB), jnp.int32),
        pltpu.VMEM_SHARED((ACC_ROWS, ROW), jnp.float32),
    ],
    compiler_params=pltpu.CompilerParams(use_tc_tiling_on_sc=False),
)
def _scatter_add(idx_hbm, ixf_hbm, out_hbm, idx_v, rows_v, sidx_v, accum_sh):
    c = lax.axis_index("c")
    s = lax.axis_index("s")
    lo = c * HALF
    zero16 = jnp.zeros((16,), jnp.float32)
    iota = lax.iota(jnp.int32, 16)

    @pl.loop(0, B)
    def _(r):
        rows_v[r, pl.ds(0, 16)] = zero16
        rows_v[r, pl.ds(16, 16)] = zero16
        rows_v[r, pl.ds(32, 16)] = zero16

    pltpu.sync_copy(rows_v.at[pl.ds(0, B)],
                    accum_sh.at[pl.ds(s * ZROWS, B)])
    pltpu.sync_copy(rows_v.at[pl.ds(0, ZROWS - B)],
                    accum_sh.at[pl.ds(s * ZROWS + B, ZROWS - B)])
    plsc.subcore_barrier()

    @pl.loop(0, NBLK)
    def _(blk):
        e0 = jnp.minimum(blk * B, E_TILE - B)
        e0g = s * E_TILE + e0
        cov = blk * B
        pltpu.sync_copy(idx_hbm.at[pl.ds(e0g, B)], idx_v)
        pltpu.sync_copy(ixf_hbm.at[pl.ds(e0g, B)], rows_v)

        @pl.loop(0, B // 16)
        def _(i):
            v = idx_v[pl.ds(i * 16, 16)]
            eid = e0 + i * 16 + iota
            m = (v >= lo) & (v < lo + HALF) & (eid >= cov)
            si = jnp.where(m, v - lo, DUMMY)
            sidx_v[i >> 3, pl.ds((i & 7) * 16, 16)] = si

        for j in range(NSUB):
            pltpu.sync_copy(rows_v.at[pl.ds(j * SUB, SUB)],
                            accum_sh.at[sidx_v.at[j]], add=True)

    plsc.subcore_barrier()
    wstart = jnp.minimum(s * W, HALF - W)
    pltpu.sync_copy(accum_sh.at[pl.ds(wstart, W)],
                    out_hbm.at[pl.ds(c * HALF + wstart, W)])


def kernel(ind_2, px, ix):
    n_atoms = px.shape[0]
    n_pairs, x_dim, c_dim = ix.shape
    idx = ind_2[:, 0]
    ixf = ix.reshape(n_pairs, x_dim * c_dim)
    out = _scatter_add(idx, ixf)
    return out.reshape(n_atoms, x_dim, c_dim)


# speedup vs baseline: 83.5818x; 1.0286x over previous
import functools

import jax
import jax.numpy as jnp
from jax import lax
from jax.experimental import pallas as pl
from jax.experimental.pallas import tpu as pltpu
from jax.experimental.pallas import tpu_sc as plsc

N_PAIRS = 1_600_000
N_ATOMS = 50_000
ROW = 48

NC = 2
NS = 16
HALF = N_ATOMS // NC
DUMMY = HALF

B = 256
NBLK_G = N_PAIRS // B
NBLK_T = 393
ACC_ROWS = 25_024
ZROWS = ACC_ROWS // NS
W = 1568

_mesh = plsc.VectorSubcoreMesh(
    core_axis_name="c", subcore_axis_name="s", num_cores=NC, num_subcores=NS)


@functools.partial(
    pl.kernel,
    out_type=jax.ShapeDtypeStruct((N_ATOMS, ROW), jnp.float32),
    mesh=_mesh,
    scratch_types=[
        pltpu.VMEM((B,), jnp.int32),
        pltpu.VMEM((B,), jnp.int32),
        pltpu.VMEM((B,), jnp.int32),
        pltpu.VMEM((B, ROW), jnp.float32),
        pltpu.VMEM((B, ROW), jnp.float32),
        pltpu.VMEM((B, ROW), jnp.float32),
        pltpu.VMEM((B,), jnp.int32),
        pltpu.VMEM((B,), jnp.int32),
        pltpu.VMEM((B,), jnp.int32),
        pltpu.VMEM_SHARED((ACC_ROWS, ROW), jnp.float32),
        pltpu.SemaphoreType.DMA,
        pltpu.SemaphoreType.DMA,
        pltpu.SemaphoreType.DMA,
        pltpu.SemaphoreType.DMA,
        pltpu.SemaphoreType.DMA,
        pltpu.SemaphoreType.DMA,
    ],
    compiler_params=pltpu.CompilerParams(use_tc_tiling_on_sc=False),
)
def _scatter_add(idx_hbm, ixf_hbm, outf_hbm,
                 idx_0, idx_1, idx_2, rows_0, rows_1, rows_2,
                 sidx_0, sidx_1, sidx_2, accum_sh,
                 in_sem0, in_sem1, in_sem2, sc_sem0, sc_sem1, sc_sem2):
    idx_bufs = (idx_0, idx_1, idx_2)
    row_bufs = (rows_0, rows_1, rows_2)
    sidx_bufs = (sidx_0, sidx_1, sidx_2)
    in_sems = (in_sem0, in_sem1, in_sem2)
    sc_sems = (sc_sem0, sc_sem1, sc_sem2)
    c = lax.axis_index("c")
    s = lax.axis_index("s")
    lo = c * HALF
    zero16 = jnp.zeros((16,), jnp.float32)

    @pl.loop(0, B)
    def _(r):
        rows_0[r, pl.ds(0, 16)] = zero16
        rows_0[r, pl.ds(16, 16)] = zero16
        rows_0[r, pl.ds(32, 16)] = zero16

    z0 = s * ZROWS
    zoff = 0
    while zoff < ZROWS:
        zlen = min(B, ZROWS - zoff)
        pltpu.sync_copy(rows_0.at[pl.ds(0, zlen)],
                        accum_sh.at[pl.ds(z0 + zoff, zlen)])
        zoff += zlen
    plsc.subcore_barrier()

    def in_start(b, q):
        gp = jnp.minimum(s + NS * b, NBLK_G - 1)
        pltpu.async_copy(idx_hbm.at[pl.ds(gp * B, B)], idx_bufs[q], in_sems[q])
        pltpu.async_copy(ixf_hbm.at[pl.ds(gp * B, B)], row_bufs[q], in_sems[q])

    def in_wait(q):
        pltpu.make_async_copy(idx_hbm.at[pl.ds(0, B)],
                              idx_bufs[q], in_sems[q]).wait()
        pltpu.make_async_copy(ixf_hbm.at[pl.ds(0, B)],
                              row_bufs[q], in_sems[q]).wait()

    def sc_drain(q):
        pltpu.make_async_copy(row_bufs[q],
                              accum_sh.at[sidx_bufs[q]], sc_sems[q]).wait()

    def phase(b, q, drain):
        in_wait(q)
        hi = lo + jnp.where((s + NS * b) < NBLK_G, HALF, 0)
        idx2 = idx_bufs[q]
        sidx = sidx_bufs[q]

        @pl.loop(0, B // 16)
        def _(i):
            v = idx2[pl.ds(i * 16, 16)]
            m = (v >= lo) & (v < hi)
            si = jnp.where(m, v - lo, DUMMY)
            sidx[pl.ds(i * 16, 16)] = si

        pltpu.async_copy(row_bufs[q], accum_sh.at[sidx], sc_sems[q], add=True)
        if drain:
            sc_drain((q + 2) % 3)
        in_start(b + 2, (q + 2) % 3)

    in_start(0, 0)
    in_start(1, 1)
    phase(0, 0, drain=False)
    phase(1, 1, drain=True)
    phase(2, 2, drain=True)

    @pl.loop(3, NBLK_T, step=3)
    def _(g):
        phase(g, 0, drain=True)
        phase(g + 1, 1, drain=True)
        phase(g + 2, 2, drain=True)

    sc_drain(2)
    in_wait(0)
    in_wait(1)
    plsc.subcore_barrier()

    wstart = jnp.minimum(s * W, HALF - W)
    pltpu.sync_copy(accum_sh.at[pl.ds(wstart, W)],
                    outf_hbm.at[pl.ds(c * HALF + wstart, W)])


def kernel(ind_2, px, ix):
    n_atoms = px.shape[0]
    n_pairs, x_dim, c_dim = ix.shape
    out = _scatter_add(ind_2[:, 0], ix.reshape(n_pairs, x_dim * c_dim))
    return out.reshape(n_atoms, x_dim, c_dim)


# speedup vs baseline: 95.3870x; 1.1412x over previous
import functools

import jax
import jax.numpy as jnp
from jax import lax
from jax.experimental import pallas as pl
from jax.experimental.pallas import tpu as pltpu
from jax.experimental.pallas import tpu_sc as plsc

N_PAIRS = 1_600_000
N_ATOMS = 50_000
ROW = 48

NC = 2
NS = 16
HALF = N_ATOMS // NC
DUMMY = HALF

K = 5
N_PAIRS_C = N_PAIRS // K
B = 256
NBLK_G = N_PAIRS_C // B
NBLK_T = 81
ACC_ROWS = 25_024
ZROWS = ACC_ROWS // NS
W = 1568

_mesh = plsc.VectorSubcoreMesh(
    core_axis_name="c", subcore_axis_name="s", num_cores=NC, num_subcores=NS)


@functools.partial(
    pl.kernel,
    out_type=jax.ShapeDtypeStruct((N_ATOMS, ROW), jnp.float32),
    mesh=_mesh,
    scratch_types=[
        pltpu.VMEM((B,), jnp.int32),
        pltpu.VMEM((B,), jnp.int32),
        pltpu.VMEM((B,), jnp.int32),
        pltpu.VMEM((B, ROW), jnp.float32),
        pltpu.VMEM((B, ROW), jnp.float32),
        pltpu.VMEM((B, ROW), jnp.float32),
        pltpu.VMEM((B,), jnp.int32),
        pltpu.VMEM((B,), jnp.int32),
        pltpu.VMEM((B,), jnp.int32),
        pltpu.VMEM_SHARED((ACC_ROWS, ROW), jnp.float32),
        pltpu.SemaphoreType.DMA,
        pltpu.SemaphoreType.DMA,
        pltpu.SemaphoreType.DMA,
        pltpu.SemaphoreType.DMA,
        pltpu.SemaphoreType.DMA,
        pltpu.SemaphoreType.DMA,
    ],
    compiler_params=pltpu.CompilerParams(use_tc_tiling_on_sc=False),
)
def _scatter_chunk(idx_hbm, ixf_hbm, acc_hbm, outf_hbm,
                 idx_0, idx_1, idx_2, rows_0, rows_1, rows_2,
                 sidx_0, sidx_1, sidx_2, accum_sh,
                 in_sem0, in_sem1, in_sem2, sc_sem0, sc_sem1, sc_sem2):
    idx_bufs = (idx_0, idx_1, idx_2)
    row_bufs = (rows_0, rows_1, rows_2)
    sidx_bufs = (sidx_0, sidx_1, sidx_2)
    in_sems = (in_sem0, in_sem1, in_sem2)
    sc_sems = (sc_sem0, sc_sem1, sc_sem2)
    c = lax.axis_index("c")
    s = lax.axis_index("s")
    lo = c * HALF

    lstart = jnp.minimum(s * W, HALF - W)
    pltpu.sync_copy(acc_hbm.at[pl.ds(c * HALF + lstart, W)],
                    accum_sh.at[pl.ds(lstart, W)])
    plsc.subcore_barrier()

    def in_start(b, q):
        gp = jnp.minimum(s + NS * b, NBLK_G - 1)
        pltpu.async_copy(idx_hbm.at[pl.ds(gp * B, B)], idx_bufs[q], in_sems[q])
        pltpu.async_copy(ixf_hbm.at[pl.ds(gp * B, B)], row_bufs[q], in_sems[q])

    def in_wait(q):
        pltpu.make_async_copy(idx_hbm.at[pl.ds(0, B)],
                              idx_bufs[q], in_sems[q]).wait()
        pltpu.make_async_copy(ixf_hbm.at[pl.ds(0, B)],
                              row_bufs[q], in_sems[q]).wait()

    def sc_drain(q):
        pltpu.make_async_copy(row_bufs[q],
                              accum_sh.at[sidx_bufs[q]], sc_sems[q]).wait()

    def phase(b, q, drain):
        in_wait(q)
        hi = lo + jnp.where((s + NS * b) < NBLK_G, HALF, 0)
        idx2 = idx_bufs[q]
        sidx = sidx_bufs[q]

        @pl.loop(0, B // 16)
        def _(i):
            v = idx2[pl.ds(i * 16, 16)]
            m = (v >= lo) & (v < hi)
            si = jnp.where(m, v - lo, DUMMY)
            sidx[pl.ds(i * 16, 16)] = si

        pltpu.async_copy(row_bufs[q], accum_sh.at[sidx], sc_sems[q], add=True)
        if drain:
            sc_drain((q + 2) % 3)
        in_start(b + 2, (q + 2) % 3)

    in_start(0, 0)
    in_start(1, 1)
    phase(0, 0, drain=False)
    phase(1, 1, drain=True)
    phase(2, 2, drain=True)

    @pl.loop(3, NBLK_T, step=3)
    def _(g):
        phase(g, 0, drain=True)
        phase(g + 1, 1, drain=True)
        phase(g + 2, 2, drain=True)

    sc_drain(2)
    in_wait(0)
    in_wait(1)
    plsc.subcore_barrier()

    wstart = jnp.minimum(s * W, HALF - W)
    pltpu.sync_copy(accum_sh.at[pl.ds(wstart, W)],
                    outf_hbm.at[pl.ds(c * HALF + wstart, W)])


def kernel(ind_2, px, ix):
    n_atoms = px.shape[0]
    n_pairs, x_dim, c_dim = ix.shape
    idx = ind_2[:, 0]
    acc = jnp.zeros((n_atoms, x_dim * c_dim), jnp.float32)
    for k in range(K):
        sl = slice(k * N_PAIRS_C, (k + 1) * N_PAIRS_C)
        acc = _scatter_chunk(idx[sl],
                             ix[sl].reshape(N_PAIRS_C, x_dim * c_dim), acc)
    return acc.reshape(n_atoms, x_dim, c_dim)
